# baseline (speedup 1.0000x reference)
import functools

import jax
import jax.numpy as jnp
from jax import lax
from jax.experimental import pallas as pl
from jax.experimental.pallas import tpu as pltpu
from jax.experimental.pallas import tpu_sc as plsc

NC = 2
NS = 16
CH = 128
BR = 10240


def _cdiv(a, b):
  return (a + b - 1) // b


def _make_deg_kernel(N, NP, K):
  WR = NP // NS
  mesh = plsc.VectorSubcoreMesh(core_axis_name="c", subcore_axis_name="s")

  @functools.partial(
      pl.kernel,
      mesh=mesh,
      out_type=jax.ShapeDtypeStruct((NC, 1, NP), jnp.float32),
      scratch_types=[
          pltpu.VMEM((K, CH), jnp.int32),
          pltpu.VMEM((CH,), jnp.int32),
          pltpu.VMEM((CH,), jnp.float32),
          pltpu.VMEM((CH,), jnp.float32),
          pltpu.VMEM_SHARED((NP,), jnp.float32),
      ],
  )
  def deg_kernel(packed_hbm, out_hbm, pk_v, dst_b, ones_v, zero_v, acc):
    c = lax.axis_index("c")
    s = lax.axis_index("s")
    pltpu.sync_copy(packed_hbm.at[c, s], pk_v)
    for i in range(CH // 16):
      ones_v[pl.ds(i * 16, 16)] = jnp.ones((16,), jnp.float32)
      zero_v[pl.ds(i * 16, 16)] = jnp.zeros((16,), jnp.float32)
    for i in range(WR // CH):
      pltpu.sync_copy(zero_v, acc.at[pl.ds(s * WR + i * CH, CH)])
    plsc.subcore_barrier()

    def body(j, carry):
      for i in range(CH // 16):
        dst_b[pl.ds(i * 16, 16)] = lax.shift_right_logical(
            pk_v[j, pl.ds(i * 16, 16)], 15)
      pltpu.sync_copy(ones_v, acc.at[dst_b], add=True)
      return carry

    lax.fori_loop(0, K, body, 0)
    plsc.subcore_barrier()
    pltpu.sync_copy(acc.at[pl.ds(s * WR, WR)],
                    out_hbm.at[c, 0, pl.ds(s * WR, WR)])

  return deg_kernel


def _make_prop_kernel(N, NP, Dc, K, split_cols):
  assert K % 2 == 1
  RPT = NP // NS
  mesh = plsc.VectorSubcoreMesh(core_axis_name="c", subcore_axis_name="s")

  @functools.partial(
      pl.kernel,
      mesh=mesh,
      out_type=jax.ShapeDtypeStruct((NC, NP, Dc), jnp.float32),
      scratch_types=[
          pltpu.VMEM((K, CH), jnp.int32),
          pltpu.VMEM((CH,), jnp.int32),
          pltpu.VMEM((CH,), jnp.int32),
          pltpu.VMEM((CH,), jnp.int32),
          pltpu.VMEM((CH,), jnp.int32),
          pltpu.VMEM((CH, Dc), jnp.float32),
          pltpu.VMEM((CH, Dc), jnp.float32),
          pltpu.VMEM_SHARED((NP, Dc), jnp.float32),
          pltpu.SemaphoreType.DMA,
          pltpu.SemaphoreType.DMA,
      ],
  )
  def prop_kernel(table_hbm, packed_hbm, out_hbm, packed_v,
                  src0, dst0, src1, dst1, rows0, rows1, acc, sem0, sem1):
    c = lax.axis_index("c")
    s = lax.axis_index("s")
    off = c * NP if split_cols else 0
    pltpu.sync_copy(packed_hbm.at[s] if split_cols else packed_hbm.at[c, s],
                    packed_v)
    base = c * NP + s * RPT if split_cols else s * RPT
    pltpu.sync_copy(table_hbm.at[pl.ds(base, RPT)], acc.at[pl.ds(s * RPT, RPT)])
    plsc.subcore_barrier()

    def unpack(j, src_b, dst_b):
      for i in range(CH // 16):
        p = packed_v[j, pl.ds(i * 16, 16)]
        src_b[pl.ds(i * 16, 16)] = (p & 0x7FFF) + off
        dst_b[pl.ds(i * 16, 16)] = lax.shift_right_logical(p, 15)

    def gather(src_b, rows, sem):
      pltpu.async_copy(table_hbm.at[src_b], rows, sem)

    def drain_g(src_b, rows, sem):
      pltpu.make_async_copy(table_hbm.at[src_b], rows, sem).wait()

    def scatter(dst_b, rows):
      pltpu.sync_copy(rows, acc.at[dst_b], add=True)

    unpack(0, src0, dst0)
    gather(src0, rows0, sem0)

    def body(k, carry):
      j0 = 2 * k
      unpack(j0 + 1, src1, dst1)
      gather(src1, rows1, sem1)
      drain_g(src0, rows0, sem0)
      scatter(dst0, rows0)
      unpack(j0 + 2, src0, dst0)
      gather(src0, rows0, sem0)
      drain_g(src1, rows1, sem1)
      scatter(dst1, rows1)
      return carry

    lax.fori_loop(0, (K - 1) // 2, body, 0)
    drain_g(src0, rows0, sem0)
    scatter(dst0, rows0)
    plsc.subcore_barrier()
    pltpu.sync_copy(acc.at[pl.ds(s * RPT, RPT)],
                    out_hbm.at[c, pl.ds(s * RPT, RPT)])

  return prop_kernel


def _scale_call(deg_p, x, N, NP, DIN):
  Dc = DIN // 2

  def body(degp_ref, x_ref, u1_ref, dinv_ref):
    deg = degp_ref[0] + degp_ref[1] + 1.0
    dv = jnp.transpose(lax.rsqrt(deg), (1, 0))
    u = x_ref[...] * dv
    u1_ref[0] = u[:, :Dc]
    u1_ref[1] = u[:, Dc:]
    dinv_ref[...] = dv

  return pl.pallas_call(
      body,
      grid=(_cdiv(NP, BR),),
      in_specs=[
          pl.BlockSpec((NC, 1, BR), lambda i: (0, 0, i)),
          pl.BlockSpec((BR, DIN), lambda i: (i, 0)),
      ],
      out_specs=[
          pl.BlockSpec((NC, BR, Dc), lambda i: (0, i, 0)),
          pl.BlockSpec((BR, 1), lambda i: (i, 0)),
      ],
      out_shape=[
          jax.ShapeDtypeStruct((NC, NP, Dc), jnp.float32),
          jax.ShapeDtypeStruct((N, 1), jnp.float32),
      ],
  )(deg_p, x)


def _mlp_call(acc1, dinv, W1, b1, W2, N, NP, DIN, DHID, DOUT):
  Dc = DIN // 2

  def body(acc_ref, dinv_ref, w1_ref, b1_ref, w2_ref, u2_ref):
    dv = dinv_ref[...]
    a0 = (acc_ref[0] * dv).astype(jnp.bfloat16)
    a1 = (acc_ref[1] * dv).astype(jnp.bfloat16)
    w1 = w1_ref[...].astype(jnp.bfloat16)
    t = (jnp.dot(a0, w1[:Dc, :], preferred_element_type=jnp.float32)
         + jnp.dot(a1, w1[Dc:, :], preferred_element_type=jnp.float32)
         + b1_ref[...])
    h = jnp.maximum(t, 0.0).astype(jnp.bfloat16)
    u2_ref[...] = jnp.dot(h, w2_ref[...].astype(jnp.bfloat16),
                          preferred_element_type=jnp.float32) * dv

  return pl.pallas_call(
      body,
      grid=(_cdiv(NP, BR),),
      in_specs=[
          pl.BlockSpec((NC, BR, Dc), lambda i: (0, i, 0)),
          pl.BlockSpec((BR, 1), lambda i: (i, 0)),
          pl.BlockSpec((DIN, DHID), lambda i: (0, 0)),
          pl.BlockSpec((1, DHID), lambda i: (0, 0)),
          pl.BlockSpec((DHID, DOUT), lambda i: (0, 0)),
      ],
      out_specs=pl.BlockSpec((BR, DOUT), lambda i: (i, 0)),
      out_shape=jax.ShapeDtypeStruct((NP, DOUT), jnp.float32),
  )(acc1, dinv, W1, b1, W2)


def _final_call(acc2, u2, dinv, b2, N, NP, DOUT):
  def body(acc_ref, u2_ref, dinv_ref, b2_ref, out_ref):
    dv = dinv_ref[...]
    o = acc_ref[0] + acc_ref[1] - u2_ref[...]
    out_ref[...] = o * dv + b2_ref[...]

  return pl.pallas_call(
      body,
      grid=(_cdiv(N, BR),),
      in_specs=[
          pl.BlockSpec((NC, BR, DOUT), lambda i: (0, i, 0)),
          pl.BlockSpec((BR, DOUT), lambda i: (i, 0)),
          pl.BlockSpec((BR, 1), lambda i: (i, 0)),
          pl.BlockSpec((1, DOUT), lambda i: (0, 0)),
      ],
      out_specs=pl.BlockSpec((BR, DOUT), lambda i: (i, 0)),
      out_shape=jax.ShapeDtypeStruct((N, DOUT), jnp.float32),
  )(acc2, u2, dinv, b2)


def kernel(x, edge_index, W1, b1, W2, b2):
  N, DIN = x.shape
  E = edge_index.shape[1]
  DHID = W1.shape[1]
  DOUT = W2.shape[1]
  src = edge_index[0]
  dst = edge_index[1]

  NP = _cdiv(N + 1, 128 * NS) * 128 * NS

  EPW = E // (NC * NS)
  NCHW = _cdiv(EPW, CH)
  if NCHW % 2 == 0:
    NCHW += 1
  padw = NCHW * CH - EPW
  packed = src | (dst << 15)
  TR = NP - N

  def trash(shape, lead):
    ar = jnp.arange(shape[-1], dtype=jnp.int32)
    return N + ((ar + 37 * lead) % TR)

  wid = (jnp.arange(NC, dtype=jnp.int32)[:, None, None] * NS
         + jnp.arange(NS, dtype=jnp.int32)[None, :, None])
  tr_w = trash((NC, NS, padw), wid)
  pk_w = jnp.concatenate(
      [packed.reshape(NC, NS, EPW), tr_w | (tr_w << 15)],
      axis=2).reshape(NC, NS, NCHW, CH)
  EPT = E // NS
  NCH = _cdiv(EPT, CH)
  if NCH % 2 == 0:
    NCH += 1
  padt = NCH * CH - EPT
  tr_t = trash((NS, padt), jnp.arange(NS, dtype=jnp.int32)[:, None])
  pk_t = jnp.concatenate(
      [packed.reshape(NS, EPT), tr_t | (tr_t << 15)],
      axis=1).reshape(NS, NCH, CH)

  deg_p = _make_deg_kernel(N, NP, NCHW)(pk_w)
  u1, dinv = _scale_call(deg_p, x, N, NP, DIN)
  acc1 = _make_prop_kernel(N, NP, DIN // 2, NCH, True)(
      u1.reshape(2 * NP, DIN // 2), pk_t)
  u2 = _mlp_call(acc1, dinv, W1, b1.reshape(1, DHID), W2, N, NP, DIN, DHID,
                 DOUT)
  acc2 = _make_prop_kernel(N, NP, DOUT, NCHW, False)(u2, pk_w)
  return _final_call(acc2, u2, dinv, b2.reshape(1, DOUT), N, NP, DOUT)


# speedup vs baseline: 1.0224x; 1.0224x over previous
import functools

import jax
import jax.numpy as jnp
from jax import lax
from jax.experimental import pallas as pl
from jax.experimental.pallas import tpu as pltpu
from jax.experimental.pallas import tpu_sc as plsc

NC = 2
NS = 16
CH = 128
BR = 5120


def _cdiv(a, b):
  return (a + b - 1) // b


def _make_deg_kernel(N, NP, K):
  WR = NP // NS
  mesh = plsc.VectorSubcoreMesh(core_axis_name="c", subcore_axis_name="s")

  @functools.partial(
      pl.kernel,
      mesh=mesh,
      out_type=jax.ShapeDtypeStruct((NC, 1, NP), jnp.float32),
      scratch_types=[
          pltpu.VMEM((K, CH), jnp.int32),
          pltpu.VMEM((CH,), jnp.int32),
          pltpu.VMEM((CH,), jnp.float32),
          pltpu.VMEM((CH,), jnp.float32),
          pltpu.VMEM_SHARED((NP,), jnp.float32),
      ],
  )
  def deg_kernel(packed_hbm, out_hbm, pk_v, dst_b, ones_v, zero_v, acc):
    c = lax.axis_index("c")
    s = lax.axis_index("s")
    pltpu.sync_copy(packed_hbm.at[c, s], pk_v)
    for i in range(CH // 16):
      ones_v[pl.ds(i * 16, 16)] = jnp.ones((16,), jnp.float32)
      zero_v[pl.ds(i * 16, 16)] = jnp.zeros((16,), jnp.float32)
    for i in range(WR // CH):
      pltpu.sync_copy(zero_v, acc.at[pl.ds(s * WR + i * CH, CH)])
    plsc.subcore_barrier()

    def body(j, carry):
      for i in range(CH // 16):
        dst_b[pl.ds(i * 16, 16)] = lax.shift_right_logical(
            pk_v[j, pl.ds(i * 16, 16)], 15)
      pltpu.sync_copy(ones_v, acc.at[dst_b], add=True)
      return carry

    lax.fori_loop(0, K, body, 0)
    plsc.subcore_barrier()
    pltpu.sync_copy(acc.at[pl.ds(s * WR, WR)],
                    out_hbm.at[c, 0, pl.ds(s * WR, WR)])

  return deg_kernel


def _make_prop_kernel(N, NP, Dc, K, split_cols):
  assert K % 2 == 1
  RPT = NP // NS
  mesh = plsc.VectorSubcoreMesh(core_axis_name="c", subcore_axis_name="s")

  @functools.partial(
      pl.kernel,
      mesh=mesh,
      out_type=jax.ShapeDtypeStruct((NC, NP, Dc), jnp.float32),
      scratch_types=[
          pltpu.VMEM((K, CH), jnp.int32),
          pltpu.VMEM((CH,), jnp.int32),
          pltpu.VMEM((CH,), jnp.int32),
          pltpu.VMEM((CH,), jnp.int32),
          pltpu.VMEM((CH,), jnp.int32),
          pltpu.VMEM((CH, Dc), jnp.float32),
          pltpu.VMEM((CH, Dc), jnp.float32),
          pltpu.VMEM_SHARED((NP, Dc), jnp.float32),
          pltpu.SemaphoreType.DMA,
          pltpu.SemaphoreType.DMA,
      ],
  )
  def prop_kernel(table_hbm, packed_hbm, out_hbm, packed_v,
                  src0, dst0, src1, dst1, rows0, rows1, acc, sem0, sem1):
    c = lax.axis_index("c")
    s = lax.axis_index("s")
    off = c * NP if split_cols else 0
    pltpu.sync_copy(packed_hbm.at[s] if split_cols else packed_hbm.at[c, s],
                    packed_v)
    base = c * NP + s * RPT if split_cols else s * RPT
    pltpu.sync_copy(table_hbm.at[pl.ds(base, RPT)], acc.at[pl.ds(s * RPT, RPT)])
    plsc.subcore_barrier()

    def unpack(j, src_b, dst_b):
      for i in range(CH // 16):
        p = packed_v[j, pl.ds(i * 16, 16)]
        src_b[pl.ds(i * 16, 16)] = (p & 0x7FFF) + off
        dst_b[pl.ds(i * 16, 16)] = lax.shift_right_logical(p, 15)

    def gather(src_b, rows, sem):
      pltpu.async_copy(table_hbm.at[src_b], rows, sem)

    def drain_g(src_b, rows, sem):
      pltpu.make_async_copy(table_hbm.at[src_b], rows, sem).wait()

    def scatter(dst_b, rows):
      pltpu.sync_copy(rows, acc.at[dst_b], add=True)

    unpack(0, src0, dst0)
    gather(src0, rows0, sem0)

    def body(k, carry):
      j0 = 2 * k
      unpack(j0 + 1, src1, dst1)
      gather(src1, rows1, sem1)
      drain_g(src0, rows0, sem0)
      scatter(dst0, rows0)
      unpack(j0 + 2, src0, dst0)
      gather(src0, rows0, sem0)
      drain_g(src1, rows1, sem1)
      scatter(dst1, rows1)
      return carry

    lax.fori_loop(0, (K - 1) // 2, body, 0)
    drain_g(src0, rows0, sem0)
    scatter(dst0, rows0)
    plsc.subcore_barrier()
    pltpu.sync_copy(acc.at[pl.ds(s * RPT, RPT)],
                    out_hbm.at[c, pl.ds(s * RPT, RPT)])

  return prop_kernel


def _scale_call(deg_p, x, N, NP, DIN):
  Dc = DIN // 2

  def body(degp_ref, x_ref, u1_ref, dinv_ref):
    deg = degp_ref[0] + degp_ref[1] + 1.0
    dv = jnp.transpose(lax.rsqrt(deg), (1, 0))
    u = x_ref[...] * dv
    u1_ref[0] = u[:, :Dc]
    u1_ref[1] = u[:, Dc:]
    dinv_ref[...] = dv

  return pl.pallas_call(
      body,
      grid=(_cdiv(NP, BR),),
      in_specs=[
          pl.BlockSpec((NC, 1, BR), lambda i: (0, 0, i)),
          pl.BlockSpec((BR, DIN), lambda i: (i, 0)),
      ],
      out_specs=[
          pl.BlockSpec((NC, BR, Dc), lambda i: (0, i, 0)),
          pl.BlockSpec((BR, 1), lambda i: (i, 0)),
      ],
      out_shape=[
          jax.ShapeDtypeStruct((NC, NP, Dc), jnp.float32),
          jax.ShapeDtypeStruct((N, 1), jnp.float32),
      ],
  )(deg_p, x)


def _mlp_call(acc1, dinv, W1, b1, W2, N, NP, DIN, DHID, DOUT):
  Dc = DIN // 2

  def body(acc_ref, dinv_ref, w1_ref, b1_ref, w2_ref, u2_ref):
    dv = dinv_ref[...]
    a0 = (acc_ref[0] * dv).astype(jnp.bfloat16)
    a1 = (acc_ref[1] * dv).astype(jnp.bfloat16)
    w1 = w1_ref[...].astype(jnp.bfloat16)
    t = (jnp.dot(a0, w1[:Dc, :], preferred_element_type=jnp.float32)
         + jnp.dot(a1, w1[Dc:, :], preferred_element_type=jnp.float32)
         + b1_ref[...])
    h = jnp.maximum(t, 0.0).astype(jnp.bfloat16)
    u2_ref[...] = jnp.dot(h, w2_ref[...].astype(jnp.bfloat16),
                          preferred_element_type=jnp.float32) * dv

  return pl.pallas_call(
      body,
      grid=(_cdiv(NP, BR),),
      in_specs=[
          pl.BlockSpec((NC, BR, Dc), lambda i: (0, i, 0)),
          pl.BlockSpec((BR, 1), lambda i: (i, 0)),
          pl.BlockSpec((DIN, DHID), lambda i: (0, 0)),
          pl.BlockSpec((1, DHID), lambda i: (0, 0)),
          pl.BlockSpec((DHID, DOUT), lambda i: (0, 0)),
      ],
      out_specs=pl.BlockSpec((BR, DOUT), lambda i: (i, 0)),
      out_shape=jax.ShapeDtypeStruct((NP, DOUT), jnp.float32),
  )(acc1, dinv, W1, b1, W2)


def _final_call(acc2, u2, dinv, b2, N, NP, DOUT):
  def body(acc_ref, u2_ref, dinv_ref, b2_ref, out_ref):
    dv = dinv_ref[...]
    o = acc_ref[0] + acc_ref[1] - u2_ref[...]
    out_ref[...] = o * dv + b2_ref[...]

  return pl.pallas_call(
      body,
      grid=(_cdiv(N, BR),),
      in_specs=[
          pl.BlockSpec((NC, BR, DOUT), lambda i: (0, i, 0)),
          pl.BlockSpec((BR, DOUT), lambda i: (i, 0)),
          pl.BlockSpec((BR, 1), lambda i: (i, 0)),
          pl.BlockSpec((1, DOUT), lambda i: (0, 0)),
      ],
      out_specs=pl.BlockSpec((BR, DOUT), lambda i: (i, 0)),
      out_shape=jax.ShapeDtypeStruct((N, DOUT), jnp.float32),
  )(acc2, u2, dinv, b2)


def kernel(x, edge_index, W1, b1, W2, b2):
  N, DIN = x.shape
  E = edge_index.shape[1]
  DHID = W1.shape[1]
  DOUT = W2.shape[1]
  src = edge_index[0]
  dst = edge_index[1]

  NP = _cdiv(N + 1, 128 * NS) * 128 * NS

  EPW = E // (NC * NS)
  NCHW = _cdiv(EPW, CH)
  if NCHW % 2 == 0:
    NCHW += 1
  padw = NCHW * CH - EPW
  packed = src | (dst << 15)
  TR = NP - N

  def trash(shape, lead):
    ar = jnp.arange(shape[-1], dtype=jnp.int32)
    return N + ((ar + 37 * lead) % TR)

  wid = (jnp.arange(NC, dtype=jnp.int32)[:, None, None] * NS
         + jnp.arange(NS, dtype=jnp.int32)[None, :, None])
  tr_w = trash((NC, NS, padw), wid)
  pk_w = jnp.concatenate(
      [packed.reshape(NC, NS, EPW), tr_w | (tr_w << 15)],
      axis=2).reshape(NC, NS, NCHW, CH)
  EPT = E // NS
  NCH = _cdiv(EPT, CH)
  if NCH % 2 == 0:
    NCH += 1
  padt = NCH * CH - EPT
  tr_t = trash((NS, padt), jnp.arange(NS, dtype=jnp.int32)[:, None])
  pk_t = jnp.concatenate(
      [packed.reshape(NS, EPT), tr_t | (tr_t << 15)],
      axis=1).reshape(NS, NCH, CH)

  deg_p = _make_deg_kernel(N, NP, NCHW)(pk_w)
  u1, dinv = _scale_call(deg_p, x, N, NP, DIN)
  acc1 = _make_prop_kernel(N, NP, DIN // 2, NCH, True)(
      u1.reshape(2 * NP, DIN // 2), pk_t)
  u2 = _mlp_call(acc1, dinv, W1, b1.reshape(1, DHID), W2, N, NP, DIN, DHID,
                 DOUT)
  acc2 = _make_prop_kernel(N, NP, DOUT, NCHW, False)(u2, pk_w)
  return _final_call(acc2, u2, dinv, b2.reshape(1, DOUT), N, NP, DOUT)
